# baseline (speedup 1.0000x reference)
import functools

import jax
import jax.numpy as jnp
from jax import lax
from jax.experimental import pallas as pl
from jax.experimental.pallas import tpu as pltpu
from jax.experimental.pallas import tpu_sc as plsc

N = 10000
D = 256
E = 160000

NPAD = 10240
EPAD = 163840
CHUNK = 128
PER_TILE = EPAD // 16
PROP_CHUNKS = PER_TILE // CHUNK
PHASES = 2
PHASE_CHUNKS = PROP_CHUNKS // PHASES
PHASE_EDGES = PER_TILE // PHASES
DEG_PER_TILE = EPAD // 32
DEG_CHUNKS = DEG_PER_TILE // CHUNK
ROWS_PER_TILE = NPAD // 16
ROW_BLKS = ROWS_PER_TILE // CHUNK

BLK = 1024
GRID = NPAD // BLK


def _sc_degree_body(dst_hbm, ones_hbm, zeros_hbm, degp_hbm, dstl, ones_v, acc):
    c = lax.axis_index("c")
    s = lax.axis_index("s")
    w = c * 16 + s
    row0 = s * ROWS_PER_TILE
    pltpu.sync_copy(zeros_hbm.at[pl.ds(row0, ROWS_PER_TILE)],
                    acc.at[pl.ds(row0, ROWS_PER_TILE)])
    pltpu.sync_copy(ones_hbm, ones_v)
    pltpu.sync_copy(dst_hbm.at[pl.ds(w * DEG_CHUNKS, DEG_CHUNKS)], dstl)
    plsc.subcore_barrier()
    for j in range(DEG_CHUNKS):
        pltpu.sync_copy(ones_v, acc.at[dstl.at[j]], add=True)
    plsc.subcore_barrier()
    pltpu.sync_copy(acc.at[pl.ds(row0, ROWS_PER_TILE)],
                    degp_hbm.at[pl.ds(c * NPAD + row0, ROWS_PER_TILE)])


def _sc_propagate_body(gtab_hbm, srcb_hbm, dst_hbm, zeros_hbm, out_hbm,
                       srcl, dstl, bufa, bufb, sema, semb, acc):
    c = lax.axis_index("c")
    s = lax.axis_index("s")
    row0 = s * ROWS_PER_TILE
    pltpu.sync_copy(zeros_hbm.at[pl.ds(row0, ROWS_PER_TILE)],
                    acc.at[pl.ds(row0, ROWS_PER_TILE)])
    plsc.subcore_barrier()
    for p in range(PHASES):
        pltpu.sync_copy(
            srcb_hbm.at[pl.ds((c * 16 + s) * PER_TILE + p * PHASE_EDGES,
                              PHASE_EDGES)], srcl)
        pltpu.sync_copy(
            dst_hbm.at[pl.ds(s * PROP_CHUNKS + p * PHASE_CHUNKS,
                             PHASE_CHUNKS)], dstl)
        cp = pltpu.async_copy(gtab_hbm.at[srcl.at[pl.ds(0, CHUNK)]], bufa, sema)
        for j in range(PHASE_CHUNKS):
            buf, nbuf, nsem = ((bufa, bufb, semb) if j % 2 == 0
                               else (bufb, bufa, sema))
            if j + 1 < PHASE_CHUNKS:
                ncp = pltpu.async_copy(
                    gtab_hbm.at[srcl.at[pl.ds((j + 1) * CHUNK, CHUNK)]],
                    nbuf, nsem)
            cp.wait()
            pltpu.sync_copy(buf, acc.at[dstl.at[j]], add=True)
            if j + 1 < PHASE_CHUNKS:
                cp = ncp
    plsc.subcore_barrier()
    pltpu.sync_copy(acc.at[pl.ds(row0, ROWS_PER_TILE)],
                    out_hbm.at[pl.ds(c * NPAD + row0, ROWS_PER_TILE)])


@functools.lru_cache(maxsize=None)
def _sc_kernels():
    mesh = plsc.VectorSubcoreMesh(core_axis_name="c", subcore_axis_name="s",
                                  num_cores=2, num_subcores=16)
    degree = pl.kernel(
        _sc_degree_body,
        out_type=jax.ShapeDtypeStruct((2 * NPAD, 128), jnp.float32),
        mesh=mesh,
        scratch_types=[
            pltpu.VMEM((DEG_CHUNKS, CHUNK), jnp.int32),
            pltpu.VMEM((CHUNK, 128), jnp.float32),
            pltpu.VMEM_SHARED((NPAD, 128), jnp.float32),
        ],
    )
    propagate = pl.kernel(
        _sc_propagate_body,
        out_type=jax.ShapeDtypeStruct((2 * NPAD, 128), jnp.float32),
        mesh=mesh,
        scratch_types=[
            pltpu.VMEM((PHASE_EDGES,), jnp.int32),
            pltpu.VMEM((PHASE_CHUNKS, CHUNK), jnp.int32),
            pltpu.VMEM((CHUNK, 128), jnp.float32),
            pltpu.VMEM((CHUNK, 128), jnp.float32),
            pltpu.SemaphoreType.DMA,
            pltpu.SemaphoreType.DMA,
            pltpu.VMEM_SHARED((NPAD, 128), jnp.float32),
        ],
    )
    return degree, propagate


def _dinv_of(degp_blk):
    deg = degp_blk[0, :, 0:1] + degp_blk[1, :, 0:1] + 1.0
    return lax.rsqrt(deg)


def _mm1_body(x_ref, w1_ref, degp_ref, gout_ref):
    dinv = _dinv_of(degp_ref[...])
    u = jnp.dot(x_ref[...], w1_ref[...], preferred_element_type=jnp.float32)
    g = u * dinv
    gout_ref[0] = g[:, :128]
    gout_ref[1] = g[:, 128:]


def _mm2_body(a_ref, g_ref, degp_ref, b1_ref, w2_ref, wrt_ref, gout_ref, lg_ref):
    dinv = _dinv_of(degp_ref[...])
    pg = jnp.concatenate([a_ref[0] + g_ref[0], a_ref[1] + g_ref[1]], axis=1)
    h1 = jnp.maximum(pg * dinv + b1_ref[...], 0.0)
    u2 = jnp.dot(h1, w2_ref[...], preferred_element_type=jnp.float32)
    g2 = u2 * dinv
    gout_ref[0] = g2[:, :128]
    gout_ref[1] = g2[:, 128:]
    lg_ref[...] = jnp.dot(h1, wrt_ref[...], preferred_element_type=jnp.float32)


def _read_body(a_ref, g_ref, degp_ref, b2_ref, wrb_ref, br_ref, lg_ref, out_ref):
    dinv = _dinv_of(degp_ref[...])
    pg = jnp.concatenate([a_ref[0] + g_ref[0], a_ref[1] + g_ref[1]], axis=1)
    h2 = jnp.maximum(pg * dinv + b2_ref[...], 0.0)
    logits = (lg_ref[...] + br_ref[...]
              + jnp.dot(h2, wrb_ref[...], preferred_element_type=jnp.float32))
    m = jnp.max(logits, axis=1, keepdims=True)
    lse = m + jnp.log(jnp.sum(jnp.exp(logits - m), axis=1, keepdims=True))
    out_ref[...] = logits - lse


_spec_split = pl.BlockSpec((2, BLK, 128), lambda i: (0, i, 0))
_spec_degp = pl.BlockSpec((2, BLK, 128), lambda i: (0, i, 0))
_spec_rows = pl.BlockSpec((BLK, 256), lambda i: (i, 0))
_spec_w = pl.BlockSpec((256, 256), lambda i: (0, 0))
_spec_vec = pl.BlockSpec((256,), lambda i: (0,))

_tc_mm1 = pl.pallas_call(
    _mm1_body,
    grid=(GRID,),
    in_specs=[_spec_rows, _spec_w, _spec_degp],
    out_specs=_spec_split,
    out_shape=jax.ShapeDtypeStruct((2, NPAD, 128), jnp.float32),
)

_tc_mm2 = pl.pallas_call(
    _mm2_body,
    grid=(GRID,),
    in_specs=[_spec_split, _spec_split, _spec_degp, _spec_vec, _spec_w, _spec_w],
    out_specs=[_spec_split, _spec_rows],
    out_shape=[jax.ShapeDtypeStruct((2, NPAD, 128), jnp.float32),
               jax.ShapeDtypeStruct((NPAD, 256), jnp.float32)],
)

_RBLK = 1000
_tc_read = pl.pallas_call(
    _read_body,
    grid=(N // _RBLK,),
    in_specs=[pl.BlockSpec((2, _RBLK, 128), lambda i: (0, i, 0)),
              pl.BlockSpec((2, _RBLK, 128), lambda i: (0, i, 0)),
              pl.BlockSpec((2, _RBLK, 128), lambda i: (0, i, 0)),
              _spec_vec, _spec_w, _spec_vec,
              pl.BlockSpec((_RBLK, 256), lambda i: (i, 0))],
    out_specs=pl.BlockSpec((_RBLK, 256), lambda i: (i, 0)),
    out_shape=jax.ShapeDtypeStruct((N, 256), jnp.float32),
)


def kernel(x, edge_index, W1, b1, W2, b2, Wr, br):
    src = edge_index[0]
    dst = edge_index[1]
    pad_e = EPAD - E
    srcp = jnp.concatenate([src, jnp.zeros((pad_e,), src.dtype)])
    dstp = jnp.concatenate([dst, jnp.full((pad_e,), N, dst.dtype)])
    srcb = jnp.concatenate([srcp, srcp + NPAD])
    dst2d = dstp.reshape(EPAD // CHUNK, CHUNK)
    xp = jnp.concatenate([x, jnp.zeros((NPAD - N, D), x.dtype)])
    ones128 = jnp.ones((CHUNK, 128), jnp.float32)
    z128 = jnp.zeros((NPAD, 128), jnp.float32)

    sc_degree, sc_propagate = _sc_kernels()
    degp = sc_degree(dst2d, ones128, z128).reshape(2, NPAD, 128)
    g1 = _tc_mm1(xp, W1, degp)
    a1 = sc_propagate(g1.reshape(2 * NPAD, 128), srcb, dst2d,
                      z128).reshape(2, NPAD, 128)
    g2, lg1 = _tc_mm2(a1, g1, degp, b1, W2, Wr[:D])
    a2 = sc_propagate(g2.reshape(2 * NPAD, 128), srcb, dst2d,
                      z128).reshape(2, NPAD, 128)
    return _tc_read(a2, g2, degp, b2, Wr[D:], br, lg1)


# speedup vs baseline: 1.0048x; 1.0048x over previous
import functools

import jax
import jax.numpy as jnp
from jax import lax
from jax.experimental import pallas as pl
from jax.experimental.pallas import tpu as pltpu
from jax.experimental.pallas import tpu_sc as plsc

N = 10000
D = 256
E = 160000

NPAD = 10240
EPAD = 163840
CHUNK = 128
PER_TILE = EPAD // 16
PROP_CHUNKS = PER_TILE // CHUNK
PHASES = 2
PHASE_CHUNKS = PROP_CHUNKS // PHASES
PHASE_EDGES = PER_TILE // PHASES
DEG_PER_TILE = EPAD // 32
DEG_CHUNKS = DEG_PER_TILE // CHUNK
ROWS_PER_TILE = NPAD // 16
ROW_BLKS = ROWS_PER_TILE // CHUNK

BLK = 1024
GRID = NPAD // BLK


def _sc_degree_body(dst_hbm, ones_hbm, zeros_hbm, degp_hbm, dstl, ones_v, acc):
    c = lax.axis_index("c")
    s = lax.axis_index("s")
    w = c * 16 + s
    row0 = s * ROWS_PER_TILE
    pltpu.sync_copy(zeros_hbm.at[pl.ds(row0, ROWS_PER_TILE)],
                    acc.at[pl.ds(row0, ROWS_PER_TILE)])
    pltpu.sync_copy(ones_hbm, ones_v)
    pltpu.sync_copy(dst_hbm.at[pl.ds(w * DEG_CHUNKS, DEG_CHUNKS)], dstl)
    plsc.subcore_barrier()
    for j in range(DEG_CHUNKS):
        pltpu.sync_copy(ones_v, acc.at[dstl.at[j]], add=True)
    plsc.subcore_barrier()
    pltpu.sync_copy(acc.at[pl.ds(row0, ROWS_PER_TILE)],
                    degp_hbm.at[pl.ds(c * NPAD + row0, ROWS_PER_TILE)])


def _sc_propagate_body(gtab_hbm, srcb_hbm, dst_hbm, zeros_hbm, out_hbm,
                       srcl, dstl, bufa, bufb, sema, semb, acc):
    c = lax.axis_index("c")
    s = lax.axis_index("s")
    row0 = s * ROWS_PER_TILE
    init = pltpu.async_copy(zeros_hbm.at[pl.ds(row0, ROWS_PER_TILE)],
                            acc.at[pl.ds(row0, ROWS_PER_TILE)], sema)
    for p in range(PHASES):
        pltpu.sync_copy(
            srcb_hbm.at[pl.ds((c * 16 + s) * PER_TILE + p * PHASE_EDGES,
                              PHASE_EDGES)], srcl)
        pltpu.sync_copy(
            dst_hbm.at[pl.ds(s * PROP_CHUNKS + p * PHASE_CHUNKS,
                             PHASE_CHUNKS)], dstl)
        if p == 0:
            init.wait()
            plsc.subcore_barrier()
        cp = pltpu.async_copy(gtab_hbm.at[srcl.at[pl.ds(0, CHUNK)]], bufa, sema)
        for j in range(PHASE_CHUNKS):
            buf, nbuf, nsem = ((bufa, bufb, semb) if j % 2 == 0
                               else (bufb, bufa, sema))
            if j + 1 < PHASE_CHUNKS:
                ncp = pltpu.async_copy(
                    gtab_hbm.at[srcl.at[pl.ds((j + 1) * CHUNK, CHUNK)]],
                    nbuf, nsem)
            cp.wait()
            pltpu.sync_copy(buf, acc.at[dstl.at[j]], add=True)
            if j + 1 < PHASE_CHUNKS:
                cp = ncp
    plsc.subcore_barrier()
    pltpu.sync_copy(acc.at[pl.ds(row0, ROWS_PER_TILE)],
                    out_hbm.at[pl.ds(c * NPAD + row0, ROWS_PER_TILE)])


@functools.lru_cache(maxsize=None)
def _sc_kernels():
    mesh = plsc.VectorSubcoreMesh(core_axis_name="c", subcore_axis_name="s",
                                  num_cores=2, num_subcores=16)
    degree = pl.kernel(
        _sc_degree_body,
        out_type=jax.ShapeDtypeStruct((2 * NPAD, 128), jnp.float32),
        mesh=mesh,
        scratch_types=[
            pltpu.VMEM((DEG_CHUNKS, CHUNK), jnp.int32),
            pltpu.VMEM((CHUNK, 128), jnp.float32),
            pltpu.VMEM_SHARED((NPAD, 128), jnp.float32),
        ],
    )
    propagate = pl.kernel(
        _sc_propagate_body,
        out_type=jax.ShapeDtypeStruct((2 * NPAD, 128), jnp.float32),
        mesh=mesh,
        scratch_types=[
            pltpu.VMEM((PHASE_EDGES,), jnp.int32),
            pltpu.VMEM((PHASE_CHUNKS, CHUNK), jnp.int32),
            pltpu.VMEM((CHUNK, 128), jnp.float32),
            pltpu.VMEM((CHUNK, 128), jnp.float32),
            pltpu.SemaphoreType.DMA,
            pltpu.SemaphoreType.DMA,
            pltpu.VMEM_SHARED((NPAD, 128), jnp.float32),
        ],
    )
    return degree, propagate


def _dinv_of(degp_blk):
    deg = degp_blk[0, :, 0:1] + degp_blk[1, :, 0:1] + 1.0
    return lax.rsqrt(deg)


def _mm1_body(x_ref, w1_ref, degp_ref, gout_ref):
    dinv = _dinv_of(degp_ref[...])
    u = jnp.dot(x_ref[...], w1_ref[...], preferred_element_type=jnp.float32)
    g = u * dinv
    gout_ref[0] = g[:, :128]
    gout_ref[1] = g[:, 128:]


def _mm2_body(a_ref, g_ref, degp_ref, b1_ref, w2_ref, wrt_ref, gout_ref, lg_ref):
    dinv = _dinv_of(degp_ref[...])
    pg = jnp.concatenate([a_ref[0] + g_ref[0], a_ref[1] + g_ref[1]], axis=1)
    h1 = jnp.maximum(pg * dinv + b1_ref[...], 0.0)
    u2 = jnp.dot(h1, w2_ref[...], preferred_element_type=jnp.float32)
    g2 = u2 * dinv
    gout_ref[0] = g2[:, :128]
    gout_ref[1] = g2[:, 128:]
    lg_ref[...] = jnp.dot(h1, wrt_ref[...], preferred_element_type=jnp.float32)


def _read_body(a_ref, g_ref, degp_ref, b2_ref, wrb_ref, br_ref, lg_ref, out_ref):
    dinv = _dinv_of(degp_ref[...])
    pg = jnp.concatenate([a_ref[0] + g_ref[0], a_ref[1] + g_ref[1]], axis=1)
    h2 = jnp.maximum(pg * dinv + b2_ref[...], 0.0)
    logits = (lg_ref[...] + br_ref[...]
              + jnp.dot(h2, wrb_ref[...], preferred_element_type=jnp.float32))
    m = jnp.max(logits, axis=1, keepdims=True)
    lse = m + jnp.log(jnp.sum(jnp.exp(logits - m), axis=1, keepdims=True))
    out_ref[...] = logits - lse


_spec_split = pl.BlockSpec((2, BLK, 128), lambda i: (0, i, 0))
_spec_degp = pl.BlockSpec((2, BLK, 128), lambda i: (0, i, 0))
_spec_rows = pl.BlockSpec((BLK, 256), lambda i: (i, 0))
_spec_w = pl.BlockSpec((256, 256), lambda i: (0, 0))
_spec_vec = pl.BlockSpec((256,), lambda i: (0,))

_tc_mm1 = pl.pallas_call(
    _mm1_body,
    grid=(GRID,),
    in_specs=[_spec_rows, _spec_w, _spec_degp],
    out_specs=_spec_split,
    out_shape=jax.ShapeDtypeStruct((2, NPAD, 128), jnp.float32),
)

_tc_mm2 = pl.pallas_call(
    _mm2_body,
    grid=(GRID,),
    in_specs=[_spec_split, _spec_split, _spec_degp, _spec_vec, _spec_w, _spec_w],
    out_specs=[_spec_split, _spec_rows],
    out_shape=[jax.ShapeDtypeStruct((2, NPAD, 128), jnp.float32),
               jax.ShapeDtypeStruct((NPAD, 256), jnp.float32)],
)

_RBLK = 1000
_tc_read = pl.pallas_call(
    _read_body,
    grid=(N // _RBLK,),
    in_specs=[pl.BlockSpec((2, _RBLK, 128), lambda i: (0, i, 0)),
              pl.BlockSpec((2, _RBLK, 128), lambda i: (0, i, 0)),
              pl.BlockSpec((2, _RBLK, 128), lambda i: (0, i, 0)),
              _spec_vec, _spec_w, _spec_vec,
              pl.BlockSpec((_RBLK, 256), lambda i: (i, 0))],
    out_specs=pl.BlockSpec((_RBLK, 256), lambda i: (i, 0)),
    out_shape=jax.ShapeDtypeStruct((N, 256), jnp.float32),
)


def kernel(x, edge_index, W1, b1, W2, b2, Wr, br):
    src = edge_index[0]
    dst = edge_index[1]
    pad_e = EPAD - E
    srcp = jnp.concatenate([src, jnp.zeros((pad_e,), src.dtype)])
    dstp = jnp.concatenate([dst, jnp.full((pad_e,), N, dst.dtype)])
    srcb = jnp.concatenate([srcp, srcp + NPAD])
    dst2d = dstp.reshape(EPAD // CHUNK, CHUNK)
    xp = jnp.concatenate([x, jnp.zeros((NPAD - N, D), x.dtype)])
    ones128 = jnp.ones((CHUNK, 128), jnp.float32)
    z128 = jnp.zeros((NPAD, 128), jnp.float32)

    sc_degree, sc_propagate = _sc_kernels()
    degp = sc_degree(dst2d, ones128, z128).reshape(2, NPAD, 128)
    g1 = _tc_mm1(xp, W1, degp)
    a1 = sc_propagate(g1.reshape(2 * NPAD, 128), srcb, dst2d,
                      z128).reshape(2, NPAD, 128)
    g2, lg1 = _tc_mm2(a1, g1, degp, b1, W2, Wr[:D])
    a2 = sc_propagate(g2.reshape(2 * NPAD, 128), srcb, dst2d,
                      z128).reshape(2, NPAD, 128)
    return _tc_read(a2, g2, degp, b2, Wr[D:], br, lg1)
